# Initial kernel scaffold; baseline (speedup 1.0000x reference)
import jax
import jax.numpy as jnp
from jax.experimental import pallas as pl


def kernel(features, edge_index, W_self1, W_neigh1, b1, W_self2, W_neigh2, b2):
    raise NotImplementedError("write your pallas kernel here")


# speedup vs baseline: 4.8527x; 4.8527x over previous
import functools

import jax
import jax.numpy as jnp
from jax import lax
from jax.experimental import pallas as pl
from jax.experimental.pallas import tpu as pltpu
from jax.experimental.pallas import tpu_sc as plsc

_CH = 128


def _sc_segment_sum(table, edge_index, zeros):
    n_nodes, d = table.shape
    e = edge_index.shape[1]
    info = plsc.get_sparse_core_info()
    nc, ns = info.num_cores, info.num_subcores
    nw = nc * ns
    n_chunks = e // _CH
    assert n_chunks * _CH == e
    kmax = pl.cdiv(n_chunks, nw)
    rpt = ((n_nodes + ns - 1) // ns + 7) // 8 * 8
    rpt_last = n_nodes - rpt * (ns - 1)
    assert rpt_last > 0 and rpt_last % 8 == 0

    mesh = plsc.VectorSubcoreMesh(core_axis_name="c", subcore_axis_name="s")

    @functools.partial(
        pl.kernel,
        out_type=jax.ShapeDtypeStruct((nc, n_nodes, d), jnp.float32),
        mesh=mesh,
        scratch_types=[
            pltpu.VMEM((2, _CH), jnp.int32),
            pltpu.VMEM((_CH, d), jnp.float32),
            pltpu.VMEM_SHARED((n_nodes, d), jnp.float32),
            pltpu.SemaphoreType.DMA,
        ],
        compiler_params=pltpu.CompilerParams(use_tc_tiling_on_sc=False),
    )
    def k(table_h, ei_h, zeros_h, out_h, ij_v, rows_v, acc_sh, gsem):
        cid = lax.axis_index("c")
        sid = lax.axis_index("s")
        wid = sid * nc + cid
        r0 = sid * rpt

        @pl.when(sid < ns - 1)
        def _():
            pltpu.sync_copy(zeros_h.at[pl.ds(r0, rpt)],
                            acc_sh.at[pl.ds(r0, rpt)])

        @pl.when(sid == ns - 1)
        def _():
            pltpu.sync_copy(zeros_h.at[pl.ds((ns - 1) * rpt, rpt_last)],
                            acc_sh.at[pl.ds((ns - 1) * rpt, rpt_last)])

        plsc.subcore_barrier()

        def do_chunk(it, carry):
            c = wid + it * nw

            @pl.when(c < n_chunks)
            def _():
                b = c * _CH
                pltpu.sync_copy(ei_h.at[:, pl.ds(b, _CH)], ij_v)
                pltpu.async_copy(table_h.at[ij_v.at[0]], rows_v, gsem).wait()
                pltpu.sync_copy(rows_v, acc_sh.at[ij_v.at[1]], add=True)
            return carry

        lax.fori_loop(0, kmax, do_chunk, 0)
        plsc.subcore_barrier()

        @pl.when(sid < ns - 1)
        def _():
            pltpu.sync_copy(acc_sh.at[pl.ds(r0, rpt)],
                            out_h.at[cid, pl.ds(r0, rpt)])

        @pl.when(sid == ns - 1)
        def _():
            pltpu.sync_copy(acc_sh.at[pl.ds((ns - 1) * rpt, rpt_last)],
                            out_h.at[cid, pl.ds((ns - 1) * rpt, rpt_last)])

    return k(table, edge_index, zeros)


def _tc_proj1(x, w_self, w_neigh_pad):
    n, _ = x.shape
    d = w_self.shape[1]
    dp = w_neigh_pad.shape[1]

    def body(x_ref, ws_ref, wn_ref, s_ref, t_ref):
        xv = x_ref[...]
        s_ref[...] = jnp.dot(xv, ws_ref[...], preferred_element_type=jnp.float32)
        t = jnp.dot(xv, wn_ref[...], preferred_element_type=jnp.float32)
        col = lax.broadcasted_iota(jnp.int32, (n, dp), 1)
        t_ref[...] = t + jnp.where(col == d, 1.0, 0.0)

    return pl.pallas_call(
        body,
        out_shape=(jax.ShapeDtypeStruct((n, d), jnp.float32),
                   jax.ShapeDtypeStruct((n, dp), jnp.float32)),
    )(x, w_self, w_neigh_pad)


def _tc_mid(s1x, acc1, b1, w_self2, w_neigh2):
    n, d = s1x.shape

    def body(s_ref, a_ref, b_ref, ws_ref, wn_ref, s2_ref, t2_ref, dinv_ref):
        acc = a_ref[0] + a_ref[1]
        deg = acc[:, d:d + 1]
        dinv = 1.0 / jnp.maximum(deg, 1.0)
        h = jnp.maximum(s_ref[...] + acc[:, :d] * dinv + b_ref[...], 0.0)
        s2_ref[...] = jnp.dot(h, ws_ref[...], preferred_element_type=jnp.float32)
        t2_ref[...] = jnp.dot(h, wn_ref[...], preferred_element_type=jnp.float32)
        dinv_ref[...] = dinv

    return pl.pallas_call(
        body,
        out_shape=(jax.ShapeDtypeStruct((n, d), jnp.float32),
                   jax.ShapeDtypeStruct((n, d), jnp.float32),
                   jax.ShapeDtypeStruct((n, 1), jnp.float32)),
    )(s1x, acc1, b1.reshape(1, d), w_self2, w_neigh2)


def _tc_z(s2h, acc2, dinv, b2):
    n, d = s2h.shape

    def body(s_ref, a_ref, dinv_ref, b_ref, z_ref):
        acc = a_ref[0] + a_ref[1]
        z_ref[...] = s_ref[...] + acc * dinv_ref[...] + b_ref[...]

    return pl.pallas_call(
        body,
        out_shape=jax.ShapeDtypeStruct((n, d), jnp.float32),
    )(s2h, acc2, dinv, b2.reshape(1, d))


def _tc_decoder(z, zt, block_rows=400):
    n, d = z.shape

    def body(z_ref, zt_ref, o_ref):
        o_ref[...] = jax.nn.sigmoid(
            jnp.dot(z_ref[...], zt_ref[...], preferred_element_type=jnp.float32))

    grid = (n // block_rows,)
    return pl.pallas_call(
        body,
        grid=grid,
        in_specs=[
            pl.BlockSpec((block_rows, d), lambda i: (i, 0)),
            pl.BlockSpec((d, n), lambda i: (0, 0)),
        ],
        out_specs=pl.BlockSpec((block_rows, n), lambda i: (i, 0)),
        out_shape=jax.ShapeDtypeStruct((n, n), jnp.float32),
    )(z, zt)


def kernel(features, edge_index, W_self1, W_neigh1, b1, W_self2, W_neigh2, b2):
    n, _ = features.shape
    d = W_self1.shape[1]
    dp = 80

    w_neigh1_pad = jnp.pad(W_neigh1, ((0, 0), (0, dp - d)))
    s1x, table1 = _tc_proj1(features, W_self1, w_neigh1_pad)

    acc1 = _sc_segment_sum(table1, edge_index, jnp.zeros((n, dp), jnp.float32))
    s2h, table2, dinv = _tc_mid(s1x, acc1, b1, W_self2, W_neigh2)

    acc2 = _sc_segment_sum(table2, edge_index, jnp.zeros((n, d), jnp.float32))
    z = _tc_z(s2h, acc2, dinv, b2)

    adj = _tc_decoder(z, z.T)
    return (z, adj)
